# baseline (speedup 1.0000x reference)
import functools

import jax
import jax.numpy as jnp
from jax import lax
from jax.experimental import pallas as pl
from jax.experimental.pallas import tpu as pltpu
from jax.experimental.pallas import tpu_sc as plsc

B = 4
T = 2048
D = 1024

_info = plsc.get_sparse_core_info()
_NC, _NS, _L = _info.num_cores, _info.num_subcores, _info.num_lanes
_NW = _NC * _NS
_NTOK = B * T
_TW = T // _NW
_CR = 16
_CPB = _TW // _CR
_NCH = B * _CPB
_NBUF = 4
_GPR = D // (2 * _L)

_mesh = plsc.VectorSubcoreMesh(core_axis_name="c", subcore_axis_name="s")


@functools.partial(
    pl.kernel,
    mesh=_mesh,
    out_type=jax.ShapeDtypeStruct((_NTOK, D), jnp.float32),
    scratch_types=[
        pltpu.VMEM((B * _TW,), jnp.int32),
        pltpu.VMEM((_TW * D // 2,), jnp.uint32),
    ] + [pltpu.VMEM((_CR, D), jnp.float32) for _ in range(_NBUF)]
      + [pltpu.SemaphoreType.DMA for _ in range(2 * _NBUF + 2)],
)
def _emb_kernel(x_hbm, emb_hbm, pos_hbm, out_hbm, idx_v, pos_v,
                buf0, buf1, buf2, buf3,
                g0, g1, g2, g3, w0, w1, w2, w3, psem, isem):
    bufs = (buf0, buf1, buf2, buf3)
    gsems = (g0, g1, g2, g3)
    wsems = (w0, w1, w2, w3)
    wid = lax.axis_index("s") * _NC + lax.axis_index("c")
    t0 = wid * _TW

    icps = [pltpu.async_copy(x_hbm.at[pl.ds(b * T + t0, _TW)],
                             idx_v.at[pl.ds(b * _TW, _TW)], isem)
            for b in range(B)]
    pcp = pltpu.async_copy(pos_hbm.at[pl.ds(t0 * (D // 2), _TW * (D // 2))],
                           pos_v, psem)
    for cp in icps:
        cp.wait()

    def start_gather(j, p):
        b, c = j >> 2, j & 3
        return pltpu.async_copy(
            emb_hbm.at[idx_v.at[pl.ds(b * _TW + _CR * c, _CR)]],
            bufs[p], gsems[p])

    gcps = {0: start_gather(0, 0), 1: start_gather(1, 1)}
    wcps = [None] * _NBUF

    pcp.wait()
    for j in range(_NCH):
        p = j % _NBUF
        b, c = j >> 2, j & 3
        gcps[j].wait()

        def add_body(r, carry):
            for v in range(_GPR):
                off = (_CR * c + r) * (D // 2) + _L * v
                w = pos_v[pl.ds(off, _L)]
                lo = lax.bitcast_convert_type(w << 16, jnp.float32)
                hi = lax.bitcast_convert_type(w & jnp.uint32(0xFFFF0000), jnp.float32)
                plsc.addupdate(bufs[p].at[r, pl.ds(2 * _L * v, _L)], lo)
                plsc.addupdate(bufs[p].at[r, pl.ds(2 * _L * v + _L, _L)], hi)
            return carry

        lax.fori_loop(0, _CR, add_body, 0)

        if j + 2 < _NCH:
            pn = (j + 2) % _NBUF
            if wcps[pn] is not None:
                wcps[pn].wait()
                wcps[pn] = None
            gcps[j + 2] = start_gather(j + 2, pn)

        wcps[p] = pltpu.async_copy(
            bufs[p], out_hbm.at[pl.ds(b * T + t0 + _CR * c, _CR)], wsems[p])
    for cp in wcps:
        if cp is not None:
            cp.wait()


def kernel(x, emb_table, pos_table):
    x_flat = x.reshape(-1).astype(jnp.int32)
    pos_bf = (pos_table.astype(jnp.bfloat16)
              .reshape(T, D // 32, 2, 16)
              .transpose(0, 1, 3, 2)
              .reshape(T * D // 2, 2))
    pos_u32 = jax.lax.bitcast_convert_type(pos_bf, jnp.uint32)
    out = _emb_kernel(x_flat, emb_table, pos_u32)
    return out.reshape(B, T, D)


# speedup vs baseline: 2.0348x; 2.0348x over previous
import functools

import jax
import jax.numpy as jnp
from jax import lax
from jax.experimental import pallas as pl
from jax.experimental.pallas import tpu as pltpu
from jax.experimental.pallas import tpu_sc as plsc

B = 4
T = 2048
D = 1024

_info = plsc.get_sparse_core_info()
_NC, _NS, _L = _info.num_cores, _info.num_subcores, _info.num_lanes
_NW = _NC * _NS
_NTOK = B * T
_TW = T // _NW
_CR = 8
_NG = _TW // _CR
_NQ = B * _NG
_NTB = 8
_VPR = D // _L

_mesh = plsc.VectorSubcoreMesh(core_axis_name="c", subcore_axis_name="s")


@functools.partial(
    pl.kernel,
    mesh=_mesh,
    out_type=jax.ShapeDtypeStruct((_NTOK, D), jnp.float32),
    scratch_types=[
        pltpu.VMEM((B * _TW,), jnp.int32),
        pltpu.VMEM((2, _CR, D), jnp.float32),
    ] + [pltpu.VMEM((_CR, D), jnp.float32) for _ in range(_NTB)]
      + [pltpu.SemaphoreType.DMA for _ in range(2 * _NTB + 3)],
)
def _emb_kernel(x_hbm, emb_hbm, pos_hbm, out_hbm, idx_v, pos_v,
                b0, b1, b2, b3, b4, b5, b6, b7,
                g0, g1, g2, g3, g4, g5, g6, g7,
                w0, w1, w2, w3, w4, w5, w6, w7, p0, p1, isem):
    bufs = (b0, b1, b2, b3, b4, b5, b6, b7)
    gsems = (g0, g1, g2, g3, g4, g5, g6, g7)
    wsems = (w0, w1, w2, w3, w4, w5, w6, w7)
    psems = (p0, p1)
    wid = lax.axis_index("s") * _NC + lax.axis_index("c")
    t0 = wid * _TW

    icps = [pltpu.async_copy(x_hbm.at[pl.ds(b * T + t0, _TW)],
                             idx_v.at[pl.ds(b * _TW, _TW)], isem)
            for b in range(B)]
    for cp in icps:
        cp.wait()

    def gather(q):
        g, k = q >> 2, q & 3
        return pltpu.async_copy(
            emb_hbm.at[idx_v.at[pl.ds(k * _TW + _CR * g, _CR)]],
            bufs[q % _NTB], gsems[q % _NTB])

    def writeback(q):
        g, k = q >> 2, q & 3
        return pltpu.async_copy(
            bufs[q % _NTB],
            out_hbm.at[pl.ds(k * T + t0 + _CR * g, _CR)], wsems[q % _NTB])

    def pos_load(g):
        return pltpu.async_copy(pos_hbm.at[pl.ds(t0 + _CR * g, _CR)],
                                pos_v.at[g % 2], psems[g % 2])

    pcps = {0: pos_load(0)}
    gcps = {q: gather(q) for q in range(4)}
    wcps = {}

    for g in range(_NG):
        if g + 1 < _NG:
            pcps[g + 1] = pos_load(g + 1)
            for m in range(4):
                qn = 4 * (g + 1) + m
                qo = qn - _NTB
                if qo >= 0:
                    wcps[qo].wait()
                gcps[qn] = gather(qn)
        for k in range(4):
            gcps[4 * g + k].wait()
        pcps[g].wait()
        ps = g % 2
        pb = [bufs[(4 * g + k) % _NTB] for k in range(4)]

        def add_body(r, carry):
            for v in range(_VPR):
                sl = pl.ds(v * _L, _L)
                pvec = pos_v[ps, r, sl]
                for k in range(4):
                    plsc.addupdate(pb[k].at[r, sl], pvec)
            return carry

        lax.fori_loop(0, _CR, add_body, 0)

        for k in range(4):
            wcps[4 * g + k] = writeback(4 * g + k)
    for q in range(_NQ - _NTB, _NQ):
        wcps[q].wait()


def kernel(x, emb_table, pos_table):
    x_flat = x.reshape(-1).astype(jnp.int32)
    out = _emb_kernel(x_flat, emb_table, pos_table)
    return out.reshape(B, T, D)


# speedup vs baseline: 2.0966x; 1.0304x over previous
import functools

import jax
import jax.numpy as jnp
from jax import lax
from jax.experimental import pallas as pl
from jax.experimental.pallas import tpu as pltpu
from jax.experimental.pallas import tpu_sc as plsc

B = 4
T = 2048
D = 1024

_info = plsc.get_sparse_core_info()
_NC, _NS, _L = _info.num_cores, _info.num_subcores, _info.num_lanes
_NW = _NC * _NS
_NTOK = B * T
_TW = T // _NW
_CR = 8
_NG = _TW // _CR
_NQ = B * _NG
_NTB = 12
_VPR = D // _L

_mesh = plsc.VectorSubcoreMesh(core_axis_name="c", subcore_axis_name="s")


@functools.partial(
    pl.kernel,
    mesh=_mesh,
    out_type=jax.ShapeDtypeStruct((_NTOK, D), jnp.float32),
    scratch_types=[
        pltpu.VMEM((B * _TW,), jnp.int32),
        pltpu.VMEM((2, _CR, D), jnp.float32),
    ] + [pltpu.VMEM((_CR, D), jnp.float32) for _ in range(_NTB)]
      + [pltpu.SemaphoreType.DMA for _ in range(2 * _NTB + 3)],
)
def _emb_kernel(x_hbm, emb_hbm, pos_hbm, out_hbm, idx_v, pos_v,
                b0, b1, b2, b3, b4, b5, b6, b7, b8, b9, b10, b11,
                g0, g1, g2, g3, g4, g5, g6, g7, g8, g9, g10, g11,
                w0, w1, w2, w3, w4, w5, w6, w7, w8, w9, w10, w11,
                p0, p1, isem):
    bufs = (b0, b1, b2, b3, b4, b5, b6, b7, b8, b9, b10, b11)
    gsems = (g0, g1, g2, g3, g4, g5, g6, g7, g8, g9, g10, g11)
    wsems = (w0, w1, w2, w3, w4, w5, w6, w7, w8, w9, w10, w11)
    psems = (p0, p1)
    wid = lax.axis_index("s") * _NC + lax.axis_index("c")
    t0 = wid * _TW

    icps = [pltpu.async_copy(x_hbm.at[pl.ds(b * T + t0, _TW)],
                             idx_v.at[pl.ds(b * _TW, _TW)], isem)
            for b in range(B)]
    for cp in icps:
        cp.wait()

    def gather(q):
        g, k = q >> 2, q & 3
        return pltpu.async_copy(
            emb_hbm.at[idx_v.at[pl.ds(k * _TW + _CR * g, _CR)]],
            bufs[q % _NTB], gsems[q % _NTB])

    def writeback(q):
        g, k = q >> 2, q & 3
        return pltpu.async_copy(
            bufs[q % _NTB],
            out_hbm.at[pl.ds(k * T + t0 + _CR * g, _CR)], wsems[q % _NTB])

    def pos_load(g):
        return pltpu.async_copy(pos_hbm.at[pl.ds(t0 + _CR * g, _CR)],
                                pos_v.at[g % 2], psems[g % 2])

    pcps = {0: pos_load(0)}
    gcps = {q: gather(q) for q in range(8)}
    wcps = {}

    for g in range(_NG):
        if g + 1 < _NG:
            pcps[g + 1] = pos_load(g + 1)
        if g + 2 < _NG:
            for m in range(4):
                qn = 4 * (g + 2) + m
                qo = qn - _NTB
                if qo >= 0:
                    wcps[qo].wait()
                gcps[qn] = gather(qn)
        for k in range(4):
            gcps[4 * g + k].wait()
        pcps[g].wait()
        ps = g % 2
        pb = [bufs[(4 * g + k) % _NTB] for k in range(4)]

        def add_body(r, carry):
            for v in range(_VPR):
                sl = pl.ds(v * _L, _L)
                pvec = pos_v[ps, r, sl]
                for k in range(4):
                    plsc.addupdate(pb[k].at[r, sl], pvec)
            return carry

        lax.fori_loop(0, _CR, add_body, 0)

        for k in range(4):
            wcps[4 * g + k] = writeback(4 * g + k)
    for q in range(_NQ - _NTB, _NQ):
        wcps[q].wait()


def kernel(x, emb_table, pos_table):
    x_flat = x.reshape(-1).astype(jnp.int32)
    out = _emb_kernel(x_flat, emb_table, pos_table)
    return out.reshape(B, T, D)
